# baseline (speedup 1.0000x reference)
import jax
import jax.numpy as jnp
from jax import lax
from jax.experimental import pallas as pl
from jax.experimental.pallas import tpu as pltpu
from jax.experimental.pallas import tpu_sc as plsc

SEQ = 66
INNER = SEQ - 2
EMB = 128
XROW = SEQ * INNER
OROW = SEQ * EMB
NCORES = 2
NSUB = 16
NW = NCORES * NSUB
LANES = 16
BPT = 32
CHP = 3
NCHUNKS = SEQ // (2 * CHP)


def _sc_body(x_hbm, tok_hbm, w_hbm, b_hbm, pos_hbm, out_hbm,
             x_v, o_v, base_v, tok_v, w_v, b_v, sin0, sin1, sout0, sout1):
    wid = lax.axis_index("s") * NCORES + lax.axis_index("c")
    bw = wid * BPT

    pltpu.sync_copy(tok_hbm, tok_v)
    pltpu.sync_copy(w_hbm, w_v)
    pltpu.sync_copy(b_hbm, b_v)
    pltpu.sync_copy(pos_hbm, base_v)
    pltpu.make_async_copy(
        x_hbm.at[pl.ds(bw, BPT), pl.ds(0, 2 * INNER * CHP)],
        x_v.at[0], sin0).start()

    lane = lax.iota(jnp.int32, LANES)
    even = (lane % 2) == 0

    def vdot(r, o):
        acc = jnp.zeros((LANES,), jnp.float32)
        for k in range(EMB // LANES):
            tv = tok_v[r, pl.ds(k * LANES, LANES)]
            wv = plsc.load_gather(w_v, [2 * (lane + k * LANES) + o])
            acc = acc + tv * wv
        return jnp.broadcast_to(jnp.sum(acc), (LANES,))

    v00 = vdot(0, 0)
    v01 = vdot(0, 1)
    v10 = vdot(1, 0)
    v11 = vdot(1, 1)
    bias_pat = plsc.load_gather(b_v, [lane % 2])
    v0_pat = jnp.where(even, v00, v01) + bias_pat
    scale_pat = jnp.where(even, v10 - v00, v11 - v01)

    @plsc.parallel_loop(0, OROW // LANES, unroll=8)
    def _fold(t):
        sl = pl.ds(t * LANES, LANES)
        base_v[sl] = base_v[sl] + v0_pat

    dup = lane // 2
    sin = (sin0, sin1)
    sout = (sout0, sout1)

    bw_h = pl.multiple_of(bw, BPT)
    CW = 2 * INNER * CHP
    NS = 2 * CHP

    def chunk_body(c, slot, prefetch, drain):
        pltpu.make_async_copy(
            x_hbm.at[pl.ds(0, BPT), pl.ds(0, CW)],
            x_v.at[slot], sin[slot]).wait()
        if prefetch:
            pltpu.make_async_copy(
                x_hbm.at[pl.ds(bw_h, BPT),
                         pl.ds(pl.multiple_of((c + 1) * CW, CW), CW)],
                x_v.at[1 - slot], sin[1 - slot]).start()
        @pl.when(drain)
        def _():
            pltpu.make_async_copy(
                o_v.at[slot], out_hbm.at[pl.ds(0, NS), pl.ds(0, BPT), :],
                sout[slot]).wait()

        slot_vec = jnp.full((LANES,), slot, jnp.int32)
        for hp in range(CHP):
            bvecs = [base_v[pl.ds((NS * c + 2 * hp) * EMB + k * LANES, LANES)]
                     for k in range(2 * EMB // LANES)]

            @plsc.parallel_loop(0, BPT, unroll=2)
            def _rows(b, hp=hp, bvecs=bvecs):
                b_vec = jnp.full((LANES,), b, jnp.int32)
                for h in range(2):
                    for k in range(EMB // LANES):
                        xg = plsc.load_gather(
                            x_v, [slot_vec, b_vec,
                                  hp * 2 * INNER + h * INNER + 8 * k + dup])
                        o_v[slot, 2 * hp + h, b, pl.ds(k * LANES, LANES)] = (
                            xg.astype(jnp.float32) * scale_pat
                            + bvecs[h * (EMB // LANES) + k])

        pltpu.make_async_copy(
            o_v.at[slot],
            out_hbm.at[pl.ds(NS * c, NS), pl.ds(bw_h, BPT), :],
            sout[slot]).start()

    def dchunk(q, carry):
        c0 = 2 * q
        chunk_body(c0, 0, True, c0 >= 2)
        chunk_body(c0 + 1, 1, True, c0 + 1 >= 2)
        return carry
    lax.fori_loop(0, (NCHUNKS - 1) // 2, dchunk, 0)
    chunk_body(NCHUNKS - 1, 0, False, True)

    for slot in range(2):
        pltpu.make_async_copy(
            o_v.at[slot], out_hbm.at[pl.ds(0, 2 * CHP), pl.ds(0, BPT), :],
            sout[slot]).wait()


def kernel(inputs, token_table, dense_W, dense_b, pos_table):
    batch = inputs.shape[0]
    x = inputs.reshape(batch, XROW)
    pos = pos_table.reshape(OROW)
    b_pad = jnp.pad(dense_b.astype(jnp.float32), (0, LANES - dense_b.shape[0]))
    w_flat = dense_W.astype(jnp.float32).reshape(2 * EMB)
    run = pl.kernel(
        _sc_body,
        out_type=jax.ShapeDtypeStruct((SEQ, batch, EMB), jnp.float32),
        mesh=plsc.VectorSubcoreMesh(core_axis_name="c", subcore_axis_name="s"),
        compiler_params=pltpu.CompilerParams(needs_layout_passes=False),
        scratch_types=[
            pltpu.VMEM((2, BPT, 2 * INNER * CHP), jnp.int32),
            pltpu.VMEM((2, 2 * CHP, BPT, EMB), jnp.float32),
            pltpu.VMEM((OROW,), jnp.float32),
            pltpu.VMEM((2, EMB), jnp.float32),
            pltpu.VMEM((2 * EMB,), jnp.float32),
            pltpu.VMEM((LANES,), jnp.float32),
            pltpu.SemaphoreType.DMA,
            pltpu.SemaphoreType.DMA,
            pltpu.SemaphoreType.DMA,
            pltpu.SemaphoreType.DMA,
        ],
    )
    out_t = run(x, token_table, w_flat, b_pad, pos)
    return jnp.transpose(out_t, (1, 0, 2))
